# Initial kernel scaffold; baseline (speedup 1.0000x reference)
import jax
import jax.numpy as jnp
from jax.experimental import pallas as pl


def kernel(t, I_grid):
    raise NotImplementedError("write your pallas kernel here")


# speedup vs baseline: 4.7339x; 4.7339x over previous
import jax
import jax.numpy as jnp
from jax import lax
from jax.experimental import pallas as pl
from jax.experimental.pallas import tpu as pltpu
from jax.experimental.pallas import tpu_sc as plsc

N_POINTS = 819200
N_GRID = 2048
N_BASIS = 64
LANES = 16
CHUNK = 512

NUM_CORES = 2
NUM_SUBCORES = 16


def _prep_body(i_ref, td_ref):
    g = i_ref[...]
    g_next = jnp.concatenate([g[1:, :], g[N_GRID - 1:, :]], axis=0)
    td_ref[...] = jnp.concatenate([g, g_next - g], axis=1)


def _prep_table(I_grid):
    return pl.pallas_call(
        _prep_body,
        out_shape=jax.ShapeDtypeStruct((N_GRID, 2 * N_BASIS), jnp.float32),
    )(I_grid)


def _make_sc_interp(n_points, n_grid, n_basis, chunk):
    nw = NUM_CORES * NUM_SUBCORES
    pts_per_w = n_points // nw
    n_iters = pts_per_w // chunk
    ns = NUM_SUBCORES

    def body(t_hbm, td_hbm, out_hbm, t_v, idx_v, w_v, g_v, o_v, sem):
        wid = lax.axis_index("c") * ns + lax.axis_index("s")
        base_w = wid * pts_per_w

        def chunk_body(it, carry):
            base = base_w + it * chunk
            pltpu.sync_copy(t_hbm.at[pl.ds(base, chunk)], t_v)

            def idx_body(j, c):
                sl = pl.ds(j * LANES, LANES)
                tt = t_v[sl]
                u = jnp.clip(tt * float(n_grid - 1), 0.0, float(n_grid - 1))
                i0 = u.astype(jnp.int32)
                idx_v[sl] = i0
                w_v[sl] = u - i0.astype(jnp.float32)
                return c

            lax.fori_loop(0, chunk // LANES, idx_body, 0)

            pltpu.async_copy(td_hbm.at[idx_v], g_v, sem).wait()

            def pt_body(p, c):
                w = w_v[pl.ds(p, LANES)][0]
                for f in range(n_basis // LANES):
                    a = g_v[p, pl.ds(f * LANES, LANES)]
                    d = g_v[p, pl.ds(n_basis + f * LANES, LANES)]
                    o_v[p, pl.ds(f * LANES, LANES)] = a + w * d
                return c

            lax.fori_loop(0, chunk, pt_body, 0)

            pltpu.sync_copy(o_v, out_hbm.at[pl.ds(base, chunk)])
            return carry

        lax.fori_loop(0, n_iters, chunk_body, 0)

    return pl.kernel(
        body,
        out_type=jax.ShapeDtypeStruct((n_points, n_basis), jnp.float32),
        mesh=plsc.VectorSubcoreMesh(core_axis_name="c", subcore_axis_name="s",
                                    num_cores=NUM_CORES,
                                    num_subcores=NUM_SUBCORES),
        scratch_types=[
            pltpu.VMEM((chunk,), jnp.float32),
            pltpu.VMEM((chunk,), jnp.int32),
            pltpu.VMEM((chunk + LANES,), jnp.float32),
            pltpu.VMEM((chunk, 2 * n_basis), jnp.float32),
            pltpu.VMEM((chunk, n_basis), jnp.float32),
            pltpu.SemaphoreType.DMA,
        ],
        compiler_params=pltpu.CompilerParams(use_tc_tiling_on_sc=False),
    )


@jax.jit
def kernel(t, I_grid):
    td = _prep_table(I_grid)
    run = _make_sc_interp(N_POINTS, N_GRID, N_BASIS, CHUNK)
    return run(t.reshape(-1), td)


# speedup vs baseline: 4.7883x; 1.0115x over previous
import jax
import jax.numpy as jnp
from jax import lax
from jax.experimental import pallas as pl
from jax.experimental.pallas import tpu as pltpu
from jax.experimental.pallas import tpu_sc as plsc

N_POINTS = 819200
N_GRID = 2048
N_BASIS = 64
LANES = 16
CHUNK = 512

NUM_CORES = 2
NUM_SUBCORES = 16


def _prep_body(i_ref, td_ref):
    g = i_ref[...]
    g_next = jnp.concatenate([g[1:, :], g[N_GRID - 1:, :]], axis=0)
    td_ref[...] = jnp.concatenate([g, g_next - g], axis=1)


def _prep_table(I_grid):
    return pl.pallas_call(
        _prep_body,
        out_shape=jax.ShapeDtypeStruct((N_GRID, 2 * N_BASIS), jnp.float32),
    )(I_grid)


def _make_sc_interp(n_points, n_grid, n_basis, chunk):
    nw = NUM_CORES * NUM_SUBCORES
    pts_per_w = n_points // nw
    n_iters = pts_per_w // chunk
    ns = NUM_SUBCORES

    def body(t_hbm, td_hbm, out_hbm, t_v, idx_v, w_v, g_v, o_v, sem):
        wid = lax.axis_index("c") * ns + lax.axis_index("s")
        base_w = wid * pts_per_w

        def chunk_body(it, carry):
            base = base_w + it * chunk
            pltpu.sync_copy(t_hbm.at[pl.ds(base, chunk)], t_v)

            def idx_body(j, c):
                sl = pl.ds(j * LANES, LANES)
                tt = t_v[sl]
                u = jnp.clip(tt * float(n_grid - 1), 0.0, float(n_grid - 1))
                i0 = u.astype(jnp.int32)
                idx_v[sl] = i0
                w_v[sl] = u - i0.astype(jnp.float32)
                return c

            lax.fori_loop(0, chunk // LANES, idx_body, 0, unroll=4)

            pltpu.async_copy(td_hbm.at[idx_v], g_v, sem).wait()

            def pt_body(p, c):
                w = w_v[pl.ds(p, LANES)][0]
                for f in range(n_basis // LANES):
                    a = g_v[p, pl.ds(f * LANES, LANES)]
                    d = g_v[p, pl.ds(n_basis + f * LANES, LANES)]
                    o_v[p, pl.ds(f * LANES, LANES)] = a + w * d
                return c

            lax.fori_loop(0, chunk, pt_body, 0, unroll=8)

            pltpu.sync_copy(o_v, out_hbm.at[pl.ds(base, chunk)])
            return carry

        lax.fori_loop(0, n_iters, chunk_body, 0)

    return pl.kernel(
        body,
        out_type=jax.ShapeDtypeStruct((n_points, n_basis), jnp.float32),
        mesh=plsc.VectorSubcoreMesh(core_axis_name="c", subcore_axis_name="s",
                                    num_cores=NUM_CORES,
                                    num_subcores=NUM_SUBCORES),
        scratch_types=[
            pltpu.VMEM((chunk,), jnp.float32),
            pltpu.VMEM((chunk,), jnp.int32),
            pltpu.VMEM((chunk + LANES,), jnp.float32),
            pltpu.VMEM((chunk, 2 * n_basis), jnp.float32),
            pltpu.VMEM((chunk, n_basis), jnp.float32),
            pltpu.SemaphoreType.DMA,
        ],
        compiler_params=pltpu.CompilerParams(use_tc_tiling_on_sc=False),
    )


@jax.jit
def kernel(t, I_grid):
    td = _prep_table(I_grid)
    run = _make_sc_interp(N_POINTS, N_GRID, N_BASIS, CHUNK)
    return run(t.reshape(-1), td)


# speedup vs baseline: 4.7968x; 1.0018x over previous
import jax
import jax.numpy as jnp
from jax import lax
from jax.experimental import pallas as pl
from jax.experimental.pallas import tpu as pltpu
from jax.experimental.pallas import tpu_sc as plsc

N_POINTS = 819200
N_GRID = 2048
N_BASIS = 64
LANES = 16
CHUNK = 512

NUM_CORES = 2
NUM_SUBCORES = 16


def _prep_body(i_ref, td_ref):
    g = i_ref[...]
    g_next = jnp.concatenate([g[1:, :], g[N_GRID - 1:, :]], axis=0)
    td_ref[...] = jnp.concatenate([g, g_next - g], axis=1)


def _prep_table(I_grid):
    return pl.pallas_call(
        _prep_body,
        out_shape=jax.ShapeDtypeStruct((N_GRID, 2 * N_BASIS), jnp.float32),
    )(I_grid)


def _make_sc_interp(n_points, n_grid, n_basis, chunk):
    nw = NUM_CORES * NUM_SUBCORES
    pts_per_w = n_points // nw
    n_iters = pts_per_w // chunk
    ns = NUM_SUBCORES

    def body(t_hbm, td_hbm, out_hbm, t_v, idx_v, w_v, g_v, o_v, sem):
        wid = lax.axis_index("c") * ns + lax.axis_index("s")
        base_w = wid * pts_per_w

        def chunk_body(it, carry):
            base = base_w + it * chunk
            pltpu.sync_copy(t_hbm.at[pl.ds(base, chunk)], t_v)

            def idx_body(j, c):
                sl = pl.ds(j * LANES, LANES)
                tt = t_v[sl]
                u = jnp.clip(tt * float(n_grid - 1), 0.0, float(n_grid - 1))
                i0 = u.astype(jnp.int32)
                idx_v[sl] = i0
                w_v[sl] = u - i0.astype(jnp.float32)
                return c

            lax.fori_loop(0, chunk // LANES, idx_body, 0, unroll=4)

            pltpu.async_copy(td_hbm.at[idx_v], g_v, sem).wait()

            def pt_body(p, c):
                w = w_v[pl.ds(p, LANES)][0]
                r = p >> 1
                cb = (p & 1) * n_basis
                for f in range(n_basis // LANES):
                    a = g_v[p, pl.ds(f * LANES, LANES)]
                    d = g_v[p, pl.ds(n_basis + f * LANES, LANES)]
                    o_v[r, pl.ds(cb + f * LANES, LANES)] = a + w * d
                return c

            lax.fori_loop(0, chunk, pt_body, 0, unroll=8)

            base2 = wid * (pts_per_w // 2) + it * (chunk // 2)
            pltpu.sync_copy(o_v, out_hbm.at[pl.ds(base2, chunk // 2)])
            return carry

        lax.fori_loop(0, n_iters, chunk_body, 0)

    return pl.kernel(
        body,
        out_type=jax.ShapeDtypeStruct((n_points // 2, 2 * n_basis), jnp.float32),
        mesh=plsc.VectorSubcoreMesh(core_axis_name="c", subcore_axis_name="s",
                                    num_cores=NUM_CORES,
                                    num_subcores=NUM_SUBCORES),
        scratch_types=[
            pltpu.VMEM((chunk,), jnp.float32),
            pltpu.VMEM((chunk,), jnp.int32),
            pltpu.VMEM((chunk + LANES,), jnp.float32),
            pltpu.VMEM((chunk, 2 * n_basis), jnp.float32),
            pltpu.VMEM((chunk // 2, 2 * n_basis), jnp.float32),
            pltpu.SemaphoreType.DMA,
        ],
        compiler_params=pltpu.CompilerParams(use_tc_tiling_on_sc=False),
    )


@jax.jit
def kernel(t, I_grid):
    td = _prep_table(I_grid)
    run = _make_sc_interp(N_POINTS, N_GRID, N_BASIS, CHUNK)
    out2 = run(t.reshape(-1), td)
    return out2.reshape(N_POINTS, N_BASIS)
